# baseline (speedup 1.0000x reference)
import functools

import jax
import jax.numpy as jnp
from jax import lax
from jax.experimental import pallas as pl
from jax.experimental.pallas import tpu as pltpu
from jax.experimental.pallas import tpu_sc as plsc

N = 10000
E = 320000
D_IN = 128
H = 64
G = 4

NC = 2
NS = 16
NP = 10240
RPT = NP // NS
CB = 100
SUB = 2
EPT = E // NS
ER = E // CB
RPC = EPT // CB
NCH = RPC // SUB
SUBD = 8
ZB = 80


def _sc_mesh():
    return plsc.VectorSubcoreMesh(core_axis_name="c", subcore_axis_name="s",
                                  num_cores=NC, num_subcores=NS)


@functools.partial(
    pl.kernel,
    out_type=(jax.ShapeDtypeStruct((NC, NP), jnp.float32),
              jax.ShapeDtypeStruct((NC, NP), jnp.float32)),
    mesh=_sc_mesh(),
    compiler_params=pltpu.CompilerParams(use_tc_tiling_on_sc=False),
    scratch_types=[
        pltpu.VMEM((SUBD, CB), jnp.int32),
        pltpu.VMEM((SUBD, CB), jnp.int32),
        pltpu.VMEM((112,), jnp.float32),
        pltpu.VMEM((RPT,), jnp.float32),
        pltpu.VMEM_SHARED((NP,), jnp.float32),
        pltpu.VMEM_SHARED((NP,), jnp.float32),
        pltpu.SemaphoreType.DMA,
    ],
)
def _sc_degrees(src_h, dst_h, dego_h, degi_h, sidx, didx, ones_v, zbuf,
                acc_o, acc_i, sem):
    c = lax.axis_index("c")
    s = lax.axis_index("s")

    def fill_ones(i, carry):
        ones_v[pl.ds(i * 16, 16)] = jnp.full((16,), 1.0, jnp.float32)
        return carry

    lax.fori_loop(0, 112 // 16, fill_ones, 0)

    def fill_zeros(i, carry):
        zbuf[pl.ds(i * 16, 16)] = jnp.zeros((16,), jnp.float32)
        return carry

    lax.fori_loop(0, RPT // 16, fill_zeros, 0)
    pltpu.sync_copy(zbuf, acc_o.at[pl.ds(s * RPT, RPT)])
    pltpu.sync_copy(zbuf, acc_i.at[pl.ds(s * RPT, RPT)])
    plsc.subcore_barrier()

    row0 = c * ER + s * RPC

    def chunk(i, carry):
        r = row0 + i * SUBD
        pltpu.sync_copy(src_h.at[pl.ds(r, SUBD)], sidx)
        pltpu.sync_copy(dst_h.at[pl.ds(r, SUBD)], didx)
        cps = []
        for j in range(SUBD):
            cps.append(pltpu.async_copy(ones_v.at[pl.ds(0, CB)],
                                        acc_o.at[sidx.at[j]], sem, add=True))
            cps.append(pltpu.async_copy(ones_v.at[pl.ds(0, CB)],
                                        acc_i.at[didx.at[j]], sem, add=True))
        for cp in cps:
            cp.wait()
        return carry

    lax.fori_loop(0, RPC // SUBD, chunk, 0)
    plsc.subcore_barrier()
    pltpu.sync_copy(acc_o.at[pl.ds(s * RPT, RPT)],
                    dego_h.at[c, pl.ds(s * RPT, RPT)])
    pltpu.sync_copy(acc_i.at[pl.ds(s * RPT, RPT)],
                    degi_h.at[c, pl.ds(s * RPT, RPT)])


@functools.partial(
    pl.kernel,
    out_type=jax.ShapeDtypeStruct((NC, NP, H), jnp.float32),
    mesh=_sc_mesh(),
    compiler_params=pltpu.CompilerParams(use_tc_tiling_on_sc=False),
    scratch_types=[
        pltpu.VMEM((SUB, CB), jnp.int32),
        pltpu.VMEM((SUB, CB), jnp.int32),
        pltpu.VMEM((SUB, CB), jnp.int32),
        pltpu.VMEM((SUB, CB), jnp.int32),
        pltpu.VMEM((SUB * CB, H), jnp.float32),
        pltpu.VMEM((SUB * CB, H), jnp.float32),
        pltpu.VMEM_SHARED((NP, H), jnp.float32),
        pltpu.VMEM_SHARED((NP, H), jnp.float32),
        pltpu.SemaphoreType.DMA,
        pltpu.SemaphoreType.DMA,
        pltpu.SemaphoreType.DMA,
        pltpu.SemaphoreType.DMA,
        pltpu.SemaphoreType.DMA,
    ],
)
def _sc_aggregate(x_h, src_h, dst_h, out_h, sidx0, didx0, sidx1, didx1,
                  rows0, rows1, xbuf, acc, gs0, gs1, ss0, ss1, xsem):
    c = lax.axis_index("c")
    s = lax.axis_index("s")

    xcp = pltpu.async_copy(x_h.at[c, pl.ds(s * RPT, RPT)],
                           xbuf.at[pl.ds(s * RPT, RPT)], xsem)

    def fill_zeros(i, carry):
        r = i // (H // 16)
        co = (i % (H // 16)) * 16
        rows0[r, pl.ds(co, 16)] = jnp.zeros((16,), jnp.float32)
        return carry

    lax.fori_loop(0, SUB * CB * (H // 16), fill_zeros, 0)

    def zero_out(t, carry):
        pltpu.sync_copy(rows0, acc.at[pl.ds(s * RPT + t * SUB * CB, SUB * CB)])
        return carry

    lax.fori_loop(0, RPT // (SUB * CB), zero_out, 0)
    rem = RPT % (SUB * CB)
    if rem:
        pltpu.sync_copy(rows0.at[pl.ds(0, rem)],
                        acc.at[pl.ds(s * RPT + RPT - rem, rem)])
    xcp.wait()
    plsc.subcore_barrier()

    row0 = c * ER + s * RPC

    def load_idx(r, sidx, didx):
        pltpu.sync_copy(src_h.at[pl.ds(r, SUB)], sidx)
        pltpu.sync_copy(dst_h.at[pl.ds(r, SUB)], didx)

    def issue_gathers(sidx, rows, sem):
        return [pltpu.async_copy(xbuf.at[sidx.at[j]],
                                 rows.at[pl.ds(j * CB, CB)], sem)
                for j in range(SUB)]

    def drain(cps):
        for cp in cps:
            cp.wait()

    def issue_scatters(didx, rows, sem):
        return [pltpu.async_copy(rows.at[pl.ds(j * CB, CB)],
                                 acc.at[didx.at[j]], sem, add=True)
                for j in range(SUB)]

    def five(k, carry):
        r = row0 + 5 * k * SUB
        load_idx(r, sidx0, didx0)
        g0 = issue_gathers(sidx0, rows0, gs0)
        load_idx(r + SUB, sidx1, didx1)
        g1 = issue_gathers(sidx1, rows1, gs1)
        drain(g0)
        s0 = issue_scatters(didx0, rows0, ss0)
        drain(s0)
        load_idx(r + 2 * SUB, sidx0, didx0)
        g0 = issue_gathers(sidx0, rows0, gs0)
        drain(g1)
        s1 = issue_scatters(didx1, rows1, ss1)
        drain(s1)
        load_idx(r + 3 * SUB, sidx1, didx1)
        g1 = issue_gathers(sidx1, rows1, gs1)
        drain(g0)
        s0 = issue_scatters(didx0, rows0, ss0)
        drain(s0)
        load_idx(r + 4 * SUB, sidx0, didx0)
        g0 = issue_gathers(sidx0, rows0, gs0)
        drain(g1)
        s1 = issue_scatters(didx1, rows1, ss1)
        drain(s1)
        drain(g0)
        s0 = issue_scatters(didx0, rows0, ss0)
        drain(s0)
        return carry

    lax.fori_loop(0, NCH // 5, five, 0)
    plsc.subcore_barrier()
    pltpu.sync_copy(acc.at[pl.ds(s * RPT, RPT)],
                    out_h.at[c, pl.ds(s * RPT, RPT)])


def _tc0_body(dego_r, degi_r, sx_r, vx_r, sw_r, vw_r, xs_r, no_r, ni_r):
    no = lax.rsqrt(jnp.maximum(dego_r[...], 1.0))
    ni = lax.rsqrt(jnp.maximum(degi_r[...], 1.0))
    no_r[...] = no
    ni_r[...] = ni
    hs = jnp.dot(sx_r[...], sw_r[...], preferred_element_type=jnp.float32)
    xs_r[0, :N, :] = hs * no[0, :N, None]
    hv = jnp.dot(vx_r[...], vw_r[...], preferred_element_type=jnp.float32)
    xs_r[1, :N, :] = hv * no[1, :N, None]


def _tc0(dego, degi, sx, vx, sw, vw):
    return pl.pallas_call(
        _tc0_body,
        out_shape=(jax.ShapeDtypeStruct((NC, NP, H), jnp.float32),
                   jax.ShapeDtypeStruct((NC, NP), jnp.float32),
                   jax.ShapeDtypeStruct((NC, NP), jnp.float32)),
    )(dego, degi, sx, vx, sw, vw)


def _tc_mid_body(agg_r, no_r, ni_r, sb_r, vb_r, sw_r, vw_r, xs_r):
    no = no_r[...]
    ni = ni_r[...]
    hs = jnp.maximum(agg_r[0] * ni[0, :, None] + sb_r[...], 0.0)
    xs_r[0] = jnp.dot(hs, sw_r[...],
                      preferred_element_type=jnp.float32) * no[0, :, None]
    hv = jnp.maximum(agg_r[1] * ni[1, :, None] + vb_r[...], 0.0)
    xs_r[1] = jnp.dot(hv, vw_r[...],
                      preferred_element_type=jnp.float32) * no[1, :, None]


def _tc_mid(agg, no, ni, sb, vb, sw, vw):
    return pl.pallas_call(
        _tc_mid_body,
        out_shape=jax.ShapeDtypeStruct((NC, NP, H), jnp.float32),
    )(agg, no, ni, sb.reshape(1, H), vb.reshape(1, H), sw, vw)


def _tc_final_body(agg_r, ni_r, sb_r, vb_r, g_r, w0_r, b0_r, w1_r, b1_r,
                   w2_r, b2_r, out_r):
    ni = ni_r[...]
    hs = jnp.maximum(agg_r[0, :N] * ni[0, :N, None] + sb_r[...], 0.0)
    hv = jnp.maximum(agg_r[1, :N] * ni[1, :N, None] + vb_r[...], 0.0)
    emb_s = jnp.mean(hs, axis=0, keepdims=True)
    emb_v = jnp.mean(hv, axis=0, keepdims=True)
    comb = jnp.concatenate([emb_s, emb_v, g_r[...]], axis=1)
    h = jnp.dot(comb, w0_r[...],
                preferred_element_type=jnp.float32) + b0_r[...]
    h = jnp.maximum(h, 0.0)
    h = jnp.maximum(
        jnp.dot(h, w1_r[...], preferred_element_type=jnp.float32) + b1_r[...],
        0.0)
    out_r[...] = (jnp.sum(h * w2_r[...], axis=1, keepdims=True)
                  + b2_r[...])


def _tc_final(agg, ni, sb, vb, g, w0, b0, w1, b1, w2, b2):
    return pl.pallas_call(
        _tc_final_body,
        out_shape=jax.ShapeDtypeStruct((1, 1), jnp.float32),
    )(agg, ni, sb.reshape(1, H), vb.reshape(1, H), g, w0,
      b0.reshape(1, -1), w1, b1.reshape(1, -1), w2.reshape(1, -1),
      b2.reshape(1, -1))


def kernel(solute_x, solute_edge_index, solvent_x, solvent_edge_index,
           global_feats,
           sol_W0, sol_b0, sol_W1, sol_b1, sol_W2, sol_b2,
           solv_W0, solv_b0, solv_W1, solv_b1, solv_W2, solv_b2,
           mlp_W0, mlp_b0, mlp_W1, mlp_b1, mlp_W2, mlp_b2):
    se = solute_edge_index.astype(jnp.int32)
    ve = solvent_edge_index.astype(jnp.int32)
    src_l = jnp.concatenate([se[0], ve[0]]).reshape(2 * ER, CB)
    dst_l = jnp.concatenate([se[1], ve[1]]).reshape(2 * ER, CB)

    dego, degi = _sc_degrees(src_l, dst_l)
    xs, no, ni = _tc0(dego, degi, solute_x, solvent_x, sol_W0, solv_W0)

    agg = _sc_aggregate(xs, src_l, dst_l)
    xs = _tc_mid(agg, no, ni, sol_b0, solv_b0, sol_W1, solv_W1)
    agg = _sc_aggregate(xs, src_l, dst_l)
    xs = _tc_mid(agg, no, ni, sol_b1, solv_b1, sol_W2, solv_W2)
    agg = _sc_aggregate(xs, src_l, dst_l)

    return _tc_final(agg, ni, sol_b2, solv_b2, global_feats,
                     mlp_W0, mlp_b0, mlp_W1, mlp_b1, mlp_W2, mlp_b2)


# speedup vs baseline: 1.4468x; 1.4468x over previous
import functools

import jax
import jax.numpy as jnp
from jax import lax
from jax.experimental import pallas as pl
from jax.experimental.pallas import tpu as pltpu
from jax.experimental.pallas import tpu_sc as plsc

N = 10000
E = 320000
D_IN = 128
H = 64
G = 4

NC = 2
NS = 16
NP = 10240
RPT = NP // NS
CB = 125
SUB = 4
EPT = E // NS
ER = E // CB
RPC = EPT // CB
NCH = RPC // SUB
SUBD = 8
ZB = 80


def _sc_mesh():
    return plsc.VectorSubcoreMesh(core_axis_name="c", subcore_axis_name="s",
                                  num_cores=NC, num_subcores=NS)


@functools.partial(
    pl.kernel,
    out_type=(jax.ShapeDtypeStruct((NC, NP), jnp.float32),
              jax.ShapeDtypeStruct((NC, NP), jnp.float32)),
    mesh=_sc_mesh(),
    compiler_params=pltpu.CompilerParams(use_tc_tiling_on_sc=False),
    scratch_types=[
        pltpu.VMEM((SUBD, CB), jnp.int32),
        pltpu.VMEM((SUBD, CB), jnp.int32),
        pltpu.VMEM((128,), jnp.float32),
        pltpu.VMEM((RPT,), jnp.float32),
        pltpu.VMEM_SHARED((NP,), jnp.float32),
        pltpu.VMEM_SHARED((NP,), jnp.float32),
        pltpu.SemaphoreType.DMA,
    ],
)
def _sc_degrees(src_h, dst_h, dego_h, degi_h, sidx, didx, ones_v, zbuf,
                acc_o, acc_i, sem):
    c = lax.axis_index("c")
    s = lax.axis_index("s")

    def fill_ones(i, carry):
        ones_v[pl.ds(i * 16, 16)] = jnp.full((16,), 1.0, jnp.float32)
        return carry

    lax.fori_loop(0, 128 // 16, fill_ones, 0)

    def fill_zeros(i, carry):
        zbuf[pl.ds(i * 16, 16)] = jnp.zeros((16,), jnp.float32)
        return carry

    lax.fori_loop(0, RPT // 16, fill_zeros, 0)
    pltpu.sync_copy(zbuf, acc_o.at[pl.ds(s * RPT, RPT)])
    pltpu.sync_copy(zbuf, acc_i.at[pl.ds(s * RPT, RPT)])
    plsc.subcore_barrier()

    row0 = c * ER + s * RPC

    def chunk(i, carry):
        r = row0 + i * SUBD
        pltpu.sync_copy(src_h.at[pl.ds(r, SUBD)], sidx)
        pltpu.sync_copy(dst_h.at[pl.ds(r, SUBD)], didx)
        cps = []
        for j in range(SUBD):
            cps.append(pltpu.async_copy(ones_v.at[pl.ds(0, CB)],
                                        acc_o.at[sidx.at[j]], sem, add=True))
            cps.append(pltpu.async_copy(ones_v.at[pl.ds(0, CB)],
                                        acc_i.at[didx.at[j]], sem, add=True))
        for cp in cps:
            cp.wait()
        return carry

    lax.fori_loop(0, RPC // SUBD, chunk, 0)
    plsc.subcore_barrier()
    pltpu.sync_copy(acc_o.at[pl.ds(s * RPT, RPT)],
                    dego_h.at[c, pl.ds(s * RPT, RPT)])
    pltpu.sync_copy(acc_i.at[pl.ds(s * RPT, RPT)],
                    degi_h.at[c, pl.ds(s * RPT, RPT)])


@functools.partial(
    pl.kernel,
    out_type=jax.ShapeDtypeStruct((NC, NP, H), jnp.float32),
    mesh=_sc_mesh(),
    compiler_params=pltpu.CompilerParams(use_tc_tiling_on_sc=False),
    scratch_types=[
        pltpu.VMEM((SUB, CB), jnp.int32),
        pltpu.VMEM((SUB, CB), jnp.int32),
        pltpu.VMEM((SUB, CB), jnp.int32),
        pltpu.VMEM((SUB, CB), jnp.int32),
        pltpu.VMEM((SUB * CB, H), jnp.float32),
        pltpu.VMEM((SUB * CB, H), jnp.float32),
        pltpu.VMEM_SHARED((NP, H), jnp.float32),
        pltpu.SemaphoreType.DMA,
        pltpu.SemaphoreType.DMA,
        pltpu.SemaphoreType.DMA,
        pltpu.SemaphoreType.DMA,
    ],
)
def _sc_aggregate(x_h, src_h, dst_h, out_h, sidx0, didx0, sidx1, didx1,
                  rows0, rows1, acc, gs0, gs1, ss0, ss1):
    c = lax.axis_index("c")
    s = lax.axis_index("s")

    def fill_zeros(i, carry):
        r = i // (H // 16)
        co = (i % (H // 16)) * 16
        rows0[r, pl.ds(co, 16)] = jnp.zeros((16,), jnp.float32)
        return carry

    lax.fori_loop(0, SUB * CB * (H // 16), fill_zeros, 0)
    pltpu.sync_copy(rows0, acc.at[pl.ds(s * RPT, SUB * CB)])
    pltpu.sync_copy(rows0.at[pl.ds(0, RPT - SUB * CB)],
                    acc.at[pl.ds(s * RPT + SUB * CB, RPT - SUB * CB)])
    plsc.subcore_barrier()

    row0 = c * ER + s * RPC

    def load_idx(r, sidx, didx):
        pltpu.sync_copy(src_h.at[pl.ds(r, SUB)], sidx)
        pltpu.sync_copy(dst_h.at[pl.ds(r, SUB)], didx)

    def issue_gathers(sidx, rows, sem):
        return [pltpu.async_copy(x_h.at[sidx.at[j]],
                                 rows.at[pl.ds(j * CB, CB)], sem)
                for j in range(SUB)]

    def drain_gathers(cps):
        for cp in cps:
            cp.wait()

    def issue_scatters(didx, rows, sem):
        return [pltpu.async_copy(rows.at[pl.ds(j * CB, CB)],
                                 acc.at[didx.at[j]], sem, add=True)
                for j in range(SUB)]

    drain = drain_gathers

    def five(k, carry):
        r = row0 + 5 * k * SUB
        load_idx(r, sidx0, didx0)
        g0 = issue_gathers(sidx0, rows0, gs0)
        load_idx(r + SUB, sidx1, didx1)
        g1 = issue_gathers(sidx1, rows1, gs1)
        drain(g0)
        s0 = issue_scatters(didx0, rows0, ss0)
        drain(s0)
        load_idx(r + 2 * SUB, sidx0, didx0)
        g0 = issue_gathers(sidx0, rows0, gs0)
        drain(g1)
        s1 = issue_scatters(didx1, rows1, ss1)
        drain(s1)
        load_idx(r + 3 * SUB, sidx1, didx1)
        g1 = issue_gathers(sidx1, rows1, gs1)
        drain(g0)
        s0 = issue_scatters(didx0, rows0, ss0)
        drain(s0)
        load_idx(r + 4 * SUB, sidx0, didx0)
        g0 = issue_gathers(sidx0, rows0, gs0)
        drain(g1)
        s1 = issue_scatters(didx1, rows1, ss1)
        drain(s1)
        drain(g0)
        s0 = issue_scatters(didx0, rows0, ss0)
        drain(s0)
        return carry

    lax.fori_loop(0, NCH // 5, five, 0)
    plsc.subcore_barrier()
    pltpu.sync_copy(acc.at[pl.ds(s * RPT, RPT)],
                    out_h.at[c, pl.ds(s * RPT, RPT)])


def _tc0_body(dego_r, degi_r, sx_r, vx_r, sw_r, vw_r, xs_r, no_r, ni_r):
    no = lax.rsqrt(jnp.maximum(dego_r[...], 1.0))
    ni = lax.rsqrt(jnp.maximum(degi_r[...], 1.0))
    no_r[...] = no
    ni_r[...] = ni
    hs = jnp.dot(sx_r[...], sw_r[...], preferred_element_type=jnp.float32)
    xs_r[0, :N, :] = hs * no[0, :N, None]
    hv = jnp.dot(vx_r[...], vw_r[...], preferred_element_type=jnp.float32)
    xs_r[1, :N, :] = hv * no[1, :N, None]


def _tc0(dego, degi, sx, vx, sw, vw):
    return pl.pallas_call(
        _tc0_body,
        out_shape=(jax.ShapeDtypeStruct((NC, NP, H), jnp.float32),
                   jax.ShapeDtypeStruct((NC, NP), jnp.float32),
                   jax.ShapeDtypeStruct((NC, NP), jnp.float32)),
    )(dego, degi, sx, vx, sw, vw)


def _tc_mid_body(agg_r, no_r, ni_r, sb_r, vb_r, sw_r, vw_r, xs_r):
    no = no_r[...]
    ni = ni_r[...]
    hs = jnp.maximum(agg_r[0] * ni[0, :, None] + sb_r[...], 0.0)
    xs_r[0] = jnp.dot(hs, sw_r[...],
                      preferred_element_type=jnp.float32) * no[0, :, None]
    hv = jnp.maximum(agg_r[1] * ni[1, :, None] + vb_r[...], 0.0)
    xs_r[1] = jnp.dot(hv, vw_r[...],
                      preferred_element_type=jnp.float32) * no[1, :, None]


def _tc_mid(agg, no, ni, sb, vb, sw, vw):
    return pl.pallas_call(
        _tc_mid_body,
        out_shape=jax.ShapeDtypeStruct((NC, NP, H), jnp.float32),
    )(agg, no, ni, sb.reshape(1, H), vb.reshape(1, H), sw, vw)


def _tc_final_body(agg_r, ni_r, sb_r, vb_r, g_r, w0_r, b0_r, w1_r, b1_r,
                   w2_r, b2_r, out_r):
    ni = ni_r[...]
    hs = jnp.maximum(agg_r[0, :N] * ni[0, :N, None] + sb_r[...], 0.0)
    hv = jnp.maximum(agg_r[1, :N] * ni[1, :N, None] + vb_r[...], 0.0)
    emb_s = jnp.mean(hs, axis=0, keepdims=True)
    emb_v = jnp.mean(hv, axis=0, keepdims=True)
    comb = jnp.concatenate([emb_s, emb_v, g_r[...]], axis=1)
    h = jnp.dot(comb, w0_r[...],
                preferred_element_type=jnp.float32) + b0_r[...]
    h = jnp.maximum(h, 0.0)
    h = jnp.maximum(
        jnp.dot(h, w1_r[...], preferred_element_type=jnp.float32) + b1_r[...],
        0.0)
    out_r[...] = (jnp.sum(h * w2_r[...], axis=1, keepdims=True)
                  + b2_r[...])


def _tc_final(agg, ni, sb, vb, g, w0, b0, w1, b1, w2, b2):
    return pl.pallas_call(
        _tc_final_body,
        out_shape=jax.ShapeDtypeStruct((1, 1), jnp.float32),
    )(agg, ni, sb.reshape(1, H), vb.reshape(1, H), g, w0,
      b0.reshape(1, -1), w1, b1.reshape(1, -1), w2.reshape(1, -1),
      b2.reshape(1, -1))


def kernel(solute_x, solute_edge_index, solvent_x, solvent_edge_index,
           global_feats,
           sol_W0, sol_b0, sol_W1, sol_b1, sol_W2, sol_b2,
           solv_W0, solv_b0, solv_W1, solv_b1, solv_W2, solv_b2,
           mlp_W0, mlp_b0, mlp_W1, mlp_b1, mlp_W2, mlp_b2):
    se = solute_edge_index.astype(jnp.int32)
    ve = solvent_edge_index.astype(jnp.int32)
    src_l = jnp.concatenate([se[0], ve[0]]).reshape(2 * ER, CB)
    dst_l = jnp.concatenate([se[1], ve[1]]).reshape(2 * ER, CB)
    src_g = jnp.concatenate([se[0], ve[0] + NP]).reshape(2 * ER, CB)

    dego, degi = _sc_degrees(src_l, dst_l)
    xs, no, ni = _tc0(dego, degi, solute_x, solvent_x, sol_W0, solv_W0)

    agg = _sc_aggregate(xs.reshape(NC * NP, H), src_g, dst_l)
    xs = _tc_mid(agg, no, ni, sol_b0, solv_b0, sol_W1, solv_W1)
    agg = _sc_aggregate(xs.reshape(NC * NP, H), src_g, dst_l)
    xs = _tc_mid(agg, no, ni, sol_b1, solv_b1, sol_W2, solv_W2)
    agg = _sc_aggregate(xs.reshape(NC * NP, H), src_g, dst_l)

    return _tc_final(agg, ni, sol_b2, solv_b2, global_feats,
                     mlp_W0, mlp_b0, mlp_W1, mlp_b1, mlp_W2, mlp_b2)
